# baseline (speedup 1.0000x reference)
import functools

import jax
import jax.numpy as jnp
from jax import lax
from jax.experimental import pallas as pl
from jax.experimental.pallas import tpu as pltpu
from jax.experimental.pallas import tpu_sc as plsc

_CH = 128


def _gather_kernel(B, V, D):
    info = plsc.get_sparse_core_info()
    NC, NS, L = info.num_cores, info.num_subcores, info.num_lanes
    NW = NC * NS
    CH = _CH
    assert D == 64 and B % (8 * NW) == 0
    b_per_w = B // NW
    n_chunks = b_per_w // CH
    mesh = plsc.VectorSubcoreMesh(core_axis_name="c", subcore_axis_name="s")

    @functools.partial(
        pl.kernel,
        mesh=mesh,
        out_type=jax.ShapeDtypeStruct((D, B), jnp.float32),
        compiler_params=pltpu.CompilerParams(
            needs_layout_passes=False, use_tc_tiling_on_sc=False
        ),
        scratch_types=[
            pltpu.VMEM((b_per_w // CH, CH), jnp.int32),
            pltpu.VMEM((D, CH), jnp.float32),
            pltpu.SemaphoreType.DMA,
        ],
    )
    def k(wflat_hbm, idx_hbm, out_hbm, idx_v, obuf, sem):
        wid = lax.axis_index("s") * NC + lax.axis_index("c")
        base = wid * b_per_w
        for r in range(b_per_w // CH):
            pltpu.sync_copy(
                idx_hbm.at[pl.ds(base + r * CH, CH)], idx_v.at[r]
            )
        for ch in range(n_chunks):
            copies = [
                pltpu.make_async_copy(
                    wflat_hbm.at[pl.ds(d * V, V)].at[idx_v.at[ch]],
                    obuf.at[d],
                    sem,
                )
                for d in range(D)
            ]
            for cp in copies:
                cp.start()
            for cp in copies:
                cp.wait()
            pltpu.sync_copy(
                obuf, out_hbm.at[:, pl.ds(base + ch * CH, CH)]
            )

    return k


def kernel(x, weight):
    B = x.shape[0]
    V, D = weight.shape
    k = _gather_kernel(B, V, D)
    wflat = weight.T.reshape(V * D)
    out_t = k(wflat, x.astype(jnp.int32))
    return out_t.T


# speedup vs baseline: 8.9956x; 8.9956x over previous
import functools

import jax
import jax.numpy as jnp
from jax import lax
from jax.experimental import pallas as pl
from jax.experimental.pallas import tpu as pltpu
from jax.experimental.pallas import tpu_sc as plsc

_CH = 256


def _gather_kernel(B, V, D):
    info = plsc.get_sparse_core_info()
    NC, NS, L = info.num_cores, info.num_subcores, info.num_lanes
    NW = NC * NS
    CH = _CH
    assert D == 64 and B % (8 * NW) == 0
    b_per_w = B // NW
    n_chunks = b_per_w // CH
    n_streams = CH // 128
    mesh = plsc.VectorSubcoreMesh(core_axis_name="c", subcore_axis_name="s")

    @functools.partial(
        pl.kernel,
        mesh=mesh,
        out_type=jax.ShapeDtypeStruct((B, 2 * D), jnp.float32),
        compiler_params=pltpu.CompilerParams(needs_layout_passes=False),
        scratch_types=[
            pltpu.VMEM((b_per_w // 128, 128), jnp.int32),
            pltpu.VMEM((CH, 2 * D), jnp.float32),
            pltpu.SemaphoreType.DMA,
        ],
    )
    def k(wpad_hbm, idx_hbm, out_hbm, idx_v, rows_v, sem):
        wid = lax.axis_index("s") * NC + lax.axis_index("c")
        base = wid * b_per_w
        for r in range(b_per_w // 128):
            pltpu.sync_copy(
                idx_hbm.at[pl.ds(base + r * 128, 128)], idx_v.at[r]
            )
        for ch in range(n_chunks):
            copies = [
                pltpu.make_async_copy(
                    wpad_hbm.at[idx_v.at[ch * n_streams + g]],
                    rows_v.at[pl.ds(g * 128, 128)],
                    sem,
                )
                for g in range(n_streams)
            ]
            for cp in copies:
                cp.start()
            for cp in copies:
                cp.wait()
            pltpu.sync_copy(
                rows_v, out_hbm.at[pl.ds(base + ch * CH, CH)]
            )

    return k


def kernel(x, weight):
    B = x.shape[0]
    V, D = weight.shape
    k = _gather_kernel(B, V, D)
    wpad = jnp.pad(weight, ((0, 0), (0, D)))
    out2 = k(wpad, x.astype(jnp.int32))
    return out2[:, :D]
